# baseline (speedup 1.0000x reference)
import jax
import jax.numpy as jnp
from jax.experimental import pallas as pl
from jax.experimental.pallas import tpu as pltpu

_NUM_CODES = 8192
_DIM = 32
_TB = 256


def _vq_block(x_ref, cb_ref, out_ref, b_ref, cbm2_ref):
    @pl.when(pl.program_id(0) == 0)
    def _():
        cb0 = cb_ref[...]
        b_ref[...] = jnp.sum(cb0 * cb0, axis=0, keepdims=True)
        cbm2_ref[...] = -2.0 * cb0

    x = x_ref[...]
    a = jnp.sum(x * x, axis=1, keepdims=True)
    m2 = jnp.dot(x, cbm2_ref[...], preferred_element_type=jnp.float32)
    norms = (a + b_ref[...]) + m2
    vmin = jnp.min(norms, axis=1, keepdims=True)
    iota = jax.lax.broadcasted_iota(jnp.int32, norms.shape, 1)
    idx = jnp.min(jnp.where(norms == vmin, iota, _NUM_CODES), axis=1)
    onehot = (iota == idx[:, None]).astype(jnp.float32)
    q = jax.lax.dot_general(onehot, cb_ref[...], (((1,), (1,)), ((), ())),
                            preferred_element_type=jnp.float32)
    out_ref[...] = x + (q - x)


def kernel(inputs, codebook):
    original_shape = inputs.shape
    x = inputs.reshape(-1, _DIM)
    n = x.shape[0]
    grid = (n // _TB,)
    out = pl.pallas_call(
        _vq_block,
        grid=grid,
        in_specs=[
            pl.BlockSpec((_TB, _DIM), lambda i: (i, 0)),
            pl.BlockSpec((_DIM, _NUM_CODES), lambda i: (0, 0)),
        ],
        out_specs=pl.BlockSpec((_TB, _DIM), lambda i: (i, 0)),
        out_shape=jax.ShapeDtypeStruct((n, _DIM), jnp.float32),
        scratch_shapes=[
            pltpu.VMEM((1, _NUM_CODES), jnp.float32),
            pltpu.VMEM((_DIM, _NUM_CODES), jnp.float32),
        ],
    )(x, codebook)
    return out.reshape(original_shape)


# speedup vs baseline: 1.3368x; 1.3368x over previous
import jax
import jax.numpy as jnp
from jax.experimental import pallas as pl
from jax.experimental.pallas import tpu as pltpu

_NUM_CODES = 8192
_DIM = 32
_TB = 256
_CH = 128


def _vq_block(x_ref, cb_ref, out_ref, b_ref, cbm2_ref):
    @pl.when(pl.program_id(0) == 0)
    def _():
        cb0 = cb_ref[...]
        b_ref[...] = jnp.sum(cb0 * cb0, axis=0, keepdims=True)
        cbm2_ref[...] = -2.0 * cb0

    x = x_ref[...]
    a = jnp.sum(x * x, axis=1, keepdims=True)
    m2 = jnp.dot(x, cbm2_ref[...], preferred_element_type=jnp.float32)
    b = b_ref[...]

    nchunks = _NUM_CODES // _CH
    run_min = jnp.full((_TB, _CH), jnp.inf, dtype=jnp.float32)
    run_chunk = jnp.zeros((_TB, _CH), dtype=jnp.int32)
    for j in range(nchunks):
        sl = slice(j * _CH, (j + 1) * _CH)
        nrm = (a + b[:, sl]) + m2[:, sl]
        lt = nrm < run_min
        run_min = jnp.where(lt, nrm, run_min)
        run_chunk = jnp.where(lt, j, run_chunk)

    gmin = jnp.min(run_min, axis=1, keepdims=True)
    lane = jax.lax.broadcasted_iota(jnp.int32, (_TB, _CH), 1)
    cand = run_chunk * _CH + lane
    idx = jnp.min(jnp.where(run_min == gmin, cand, _NUM_CODES), axis=1)

    iota = jax.lax.broadcasted_iota(jnp.int32, (_TB, _NUM_CODES), 1)
    onehot = (iota == idx[:, None]).astype(jnp.float32)
    q = jax.lax.dot_general(onehot, cb_ref[...], (((1,), (1,)), ((), ())),
                            preferred_element_type=jnp.float32)
    out_ref[...] = x + (q - x)


def kernel(inputs, codebook):
    original_shape = inputs.shape
    x = inputs.reshape(-1, _DIM)
    n = x.shape[0]
    grid = (n // _TB,)
    out = pl.pallas_call(
        _vq_block,
        grid=grid,
        in_specs=[
            pl.BlockSpec((_TB, _DIM), lambda i: (i, 0)),
            pl.BlockSpec((_DIM, _NUM_CODES), lambda i: (0, 0)),
        ],
        out_specs=pl.BlockSpec((_TB, _DIM), lambda i: (i, 0)),
        out_shape=jax.ShapeDtypeStruct((n, _DIM), jnp.float32),
        scratch_shapes=[
            pltpu.VMEM((1, _NUM_CODES), jnp.float32),
            pltpu.VMEM((_DIM, _NUM_CODES), jnp.float32),
        ],
    )(x, codebook)
    return out.reshape(original_shape)


# speedup vs baseline: 2.1441x; 1.6039x over previous
import functools

import jax
import jax.numpy as jnp
from jax import lax
from jax.experimental import pallas as pl
from jax.experimental.pallas import tpu as pltpu
from jax.experimental.pallas import tpu_sc as plsc

_NUM_CODES = 8192
_DIM = 32
_TB = 256
_CH = 128


def _vq_argmin_block(x_ref, cb_ref, idx_ref, b_ref, cbm2_ref):
    @pl.when(pl.program_id(0) == 0)
    def _():
        cb0 = cb_ref[...]
        b_ref[...] = jnp.sum(cb0 * cb0, axis=0, keepdims=True)
        cbm2_ref[...] = -2.0 * cb0

    x = x_ref[...]
    a = jnp.sum(x * x, axis=1, keepdims=True)
    m2 = jnp.dot(x, cbm2_ref[...], preferred_element_type=jnp.float32)
    b = b_ref[...]

    nchunks = _NUM_CODES // _CH
    run_min = jnp.full((_TB, _CH), jnp.inf, dtype=jnp.float32)
    run_chunk = jnp.zeros((_TB, _CH), dtype=jnp.int32)
    for j in range(nchunks):
        sl = slice(j * _CH, (j + 1) * _CH)
        nrm = (a + b[:, sl]) + m2[:, sl]
        lt = nrm < run_min
        run_min = jnp.where(lt, nrm, run_min)
        run_chunk = jnp.where(lt, j, run_chunk)

    gmin = jnp.min(run_min, axis=1, keepdims=True)
    lane = jax.lax.broadcasted_iota(jnp.int32, (_TB, _CH), 1)
    cand = run_chunk * _CH + lane
    idx = jnp.min(jnp.where(run_min == gmin, cand, _NUM_CODES), axis=1)
    idx_ref[...] = idx.reshape(1, 1, _TB)


def _tc_argmin(x, codebook):
    n = x.shape[0]
    grid = (n // _TB,)
    idx = pl.pallas_call(
        _vq_argmin_block,
        grid=grid,
        in_specs=[
            pl.BlockSpec((_TB, _DIM), lambda i: (i, 0)),
            pl.BlockSpec((_DIM, _NUM_CODES), lambda i: (0, 0)),
        ],
        out_specs=pl.BlockSpec((1, 1, _TB), lambda i: (i, 0, 0)),
        out_shape=jax.ShapeDtypeStruct((n // _TB, 1, _TB), jnp.int32),
        scratch_shapes=[
            pltpu.VMEM((1, _NUM_CODES), jnp.float32),
            pltpu.VMEM((_DIM, _NUM_CODES), jnp.float32),
        ],
    )(x, codebook)
    return idx.reshape(n)


def _sc_gather(table, idx):
    B = idx.shape[0]
    info = plsc.get_sparse_core_info()
    nw = info.num_cores * info.num_subcores
    b_per_w = B // nw
    nseg = b_per_w // _CH
    idx3 = idx.reshape(nw, nseg, _CH)
    mesh = plsc.VectorSubcoreMesh(core_axis_name="c", subcore_axis_name="s")

    @functools.partial(
        pl.kernel, mesh=mesh,
        compiler_params=pltpu.CompilerParams(use_tc_tiling_on_sc=False),
        out_type=jax.ShapeDtypeStruct((B, _DIM), jnp.float32),
        scratch_types=[
            pltpu.VMEM((nseg, _CH), jnp.int32),
            pltpu.VMEM((b_per_w, _DIM), jnp.float32),
            pltpu.SemaphoreType.DMA,
        ],
    )
    def k(table_hbm, idx_hbm, out_hbm, idx_v, rows_v, sem):
        wid = lax.axis_index("s") * info.num_cores + lax.axis_index("c")
        pltpu.sync_copy(idx_hbm.at[wid], idx_v)
        copies = [
            pltpu.async_copy(
                table_hbm.at[idx_v.at[s]],
                rows_v.at[pl.ds(s * _CH, _CH)],
                sem,
            )
            for s in range(nseg)
        ]
        for c in copies:
            c.wait()
        pltpu.sync_copy(rows_v, out_hbm.at[pl.ds(wid * b_per_w, b_per_w)])

    return k(table, idx3)


def kernel(inputs, codebook):
    original_shape = inputs.shape
    x = inputs.reshape(-1, _DIM)
    idx = _tc_argmin(x, codebook)
    q = _sc_gather(codebook.T, idx)
    return q.reshape(original_shape)
